# baseline (speedup 1.0000x reference)
import functools

import jax
import jax.numpy as jnp
from jax.experimental import pallas as pl
from jax.experimental.pallas import tpu as pltpu

_NE = 8
_R = 64
_SCALING = 32.0 / 64.0
_BM = 128
_NEG = -1e30


def _body(x_ref, wb_ref, wg_ref, aall_ref, bcat_ref, out_ref, *, ne, r):
    xb = x_ref[...].astype(jnp.bfloat16)
    ner = ne * r
    bm = xb.shape[0]
    logits = jax.lax.dot_general(
        xb, wg_ref[...], (((1,), (1,)), ((), ())),
        preferred_element_type=jnp.float32)
    idx = jax.lax.broadcasted_iota(jnp.int32, (bm, ne), 1)
    m1 = jnp.max(logits, axis=1, keepdims=True)
    am1 = jnp.min(jnp.where(logits == m1, idx, ne), axis=1, keepdims=True)
    oh1 = idx == am1
    neg = jnp.where(oh1, _NEG, logits)
    m2 = jnp.max(neg, axis=1, keepdims=True)
    am2 = jnp.min(jnp.where(neg == m2, idx, ne), axis=1, keepdims=True)
    oh2 = idx == am2
    p1 = 1.0 / (1.0 + jnp.exp(m2 - m1))
    p2 = 1.0 - p1
    w = jnp.where(oh1, p1, 0.0) + jnp.where(oh2, p2, 0.0)
    col_e = jax.lax.broadcasted_iota(jnp.int32, (ne, ner), 1) // r
    row_e = jax.lax.broadcasted_iota(jnp.int32, (ne, ner), 0)
    expand = (col_e == row_e).astype(jnp.float32)
    wexp = jnp.dot(w * _SCALING, expand,
                   preferred_element_type=jnp.float32)
    h = jax.lax.dot_general(
        xb, aall_ref[...], (((1,), (1,)), ((), ())),
        preferred_element_type=jnp.float32)
    hs = (h * wexp).astype(jnp.bfloat16)
    acc = jax.lax.dot_general(
        xb, wb_ref[...], (((1,), (1,)), ((), ())),
        preferred_element_type=jnp.float32)
    acc += jnp.dot(hs, bcat_ref[...], preferred_element_type=jnp.float32)
    out_ref[...] = acc


@functools.partial(jax.jit, static_argnames=("bm", "interpret"))
def _mixlora(x, wb, wg, aall, bcat, bm=_BM, interpret=False):
    ntok, d = x.shape
    ne = wg.shape[0]
    ner = aall.shape[0]
    r = ner // ne
    grid = (ntok // bm,)
    return pl.pallas_call(
        functools.partial(_body, ne=ne, r=r),
        grid=grid,
        in_specs=[
            pl.BlockSpec((bm, d), lambda m: (m, 0)),
            pl.BlockSpec((d, d), lambda m: (0, 0)),
            pl.BlockSpec((ne, d), lambda m: (0, 0)),
            pl.BlockSpec((ner, d), lambda m: (0, 0)),
            pl.BlockSpec((ner, d), lambda m: (0, 0)),
        ],
        out_specs=pl.BlockSpec((bm, d), lambda m: (m, 0)),
        out_shape=jax.ShapeDtypeStruct((ntok, d), jnp.float32),
        compiler_params=pltpu.CompilerParams(
            dimension_semantics=("arbitrary",)),
        interpret=interpret,
    )(x, wb, wg, aall, bcat)


def kernel(x, W_base, W_gate, A, B):
    ne, r, d = A.shape
    wb = W_base.astype(jnp.bfloat16)
    wg = W_gate.astype(jnp.bfloat16)
    aall = A.reshape(ne * r, d).astype(jnp.bfloat16)
    bcat = B.transpose(0, 2, 1).reshape(ne * r, d).astype(jnp.bfloat16)
    return _mixlora(x, wb, wg, aall, bcat)


# speedup vs baseline: 2.2780x; 2.2780x over previous
import functools

import jax
import jax.numpy as jnp
from jax.experimental import pallas as pl
from jax.experimental.pallas import tpu as pltpu

_NE = 8
_R = 64
_SCALING = 32.0 / 64.0
_BMG = 512
_BM = 2048
_BN = 256
_NEG = -1e30


def _gate_body(x_ref, wg_ref, aall_ref, x16_ref, hs_ref, *, ne, r):
    xb = x_ref[...].astype(jnp.bfloat16)
    x16_ref[...] = xb
    ner = ne * r
    bm = xb.shape[0]
    logits = jax.lax.dot_general(
        xb, wg_ref[...], (((1,), (1,)), ((), ())),
        preferred_element_type=jnp.float32)
    idx = jax.lax.broadcasted_iota(jnp.int32, (bm, ne), 1)
    m1 = jnp.max(logits, axis=1, keepdims=True)
    am1 = jnp.min(jnp.where(logits == m1, idx, ne), axis=1, keepdims=True)
    oh1 = idx == am1
    neg = jnp.where(oh1, _NEG, logits)
    m2 = jnp.max(neg, axis=1, keepdims=True)
    am2 = jnp.min(jnp.where(neg == m2, idx, ne), axis=1, keepdims=True)
    oh2 = idx == am2
    p1 = 1.0 / (1.0 + jnp.exp(m2 - m1))
    p2 = 1.0 - p1
    w = jnp.where(oh1, p1, 0.0) + jnp.where(oh2, p2, 0.0)
    col_e = jax.lax.broadcasted_iota(jnp.int32, (ne, ner), 1) // r
    row_e = jax.lax.broadcasted_iota(jnp.int32, (ne, ner), 0)
    expand = (col_e == row_e).astype(jnp.float32)
    wexp = jnp.dot(w * _SCALING, expand,
                   preferred_element_type=jnp.float32)
    h = jax.lax.dot_general(
        xb, aall_ref[...], (((1,), (1,)), ((), ())),
        preferred_element_type=jnp.float32)
    hs_ref[...] = (h * wexp).astype(jnp.bfloat16)


def _mm_body(x16_ref, wb_ref, hs_ref, bcat_ref, out_ref):
    acc = jax.lax.dot_general(
        x16_ref[...], wb_ref[...], (((1,), (1,)), ((), ())),
        preferred_element_type=jnp.float32)
    acc += jnp.dot(hs_ref[...], bcat_ref[...],
                   preferred_element_type=jnp.float32)
    out_ref[...] = acc


@functools.partial(jax.jit, static_argnames=("bmg", "bm", "bn", "interpret"))
def _mixlora(x, wb, wg, aall, bcat, bmg=_BMG, bm=_BM, bn=_BN,
             interpret=False):
    ntok, d = x.shape
    ne = wg.shape[0]
    ner = aall.shape[0]
    r = ner // ne
    x16, hs = pl.pallas_call(
        functools.partial(_gate_body, ne=ne, r=r),
        grid=(ntok // bmg,),
        in_specs=[
            pl.BlockSpec((bmg, d), lambda m: (m, 0)),
            pl.BlockSpec((ne, d), lambda m: (0, 0)),
            pl.BlockSpec((ner, d), lambda m: (0, 0)),
        ],
        out_specs=[
            pl.BlockSpec((bmg, d), lambda m: (m, 0)),
            pl.BlockSpec((bmg, ner), lambda m: (m, 0)),
        ],
        out_shape=[
            jax.ShapeDtypeStruct((ntok, d), jnp.bfloat16),
            jax.ShapeDtypeStruct((ntok, ner), jnp.bfloat16),
        ],
        compiler_params=pltpu.CompilerParams(
            dimension_semantics=("arbitrary",)),
        interpret=interpret,
    )(x, wg, aall)
    return pl.pallas_call(
        _mm_body,
        grid=(ntok // bm, d // bn),
        in_specs=[
            pl.BlockSpec((bm, d), lambda m, n: (m, 0)),
            pl.BlockSpec((bn, d), lambda m, n: (n, 0)),
            pl.BlockSpec((bm, ner), lambda m, n: (m, 0)),
            pl.BlockSpec((ner, bn), lambda m, n: (0, n)),
        ],
        out_specs=pl.BlockSpec((bm, bn), lambda m, n: (m, n)),
        out_shape=jax.ShapeDtypeStruct((ntok, d), jnp.float32),
        compiler_params=pltpu.CompilerParams(
            dimension_semantics=("arbitrary", "arbitrary")),
        interpret=interpret,
    )(x16, wb, hs, bcat)


def kernel(x, W_base, W_gate, A, B):
    ne, r, d = A.shape
    wb = W_base.astype(jnp.bfloat16)
    wg = W_gate.astype(jnp.bfloat16)
    aall = A.reshape(ne * r, d).astype(jnp.bfloat16)
    bcat = B.transpose(0, 2, 1).reshape(ne * r, d).astype(jnp.bfloat16)
    return _mixlora(x, wb, wg, aall, bcat)


# speedup vs baseline: 2.4319x; 1.0676x over previous
import functools

import jax
import jax.numpy as jnp
from jax.experimental import pallas as pl
from jax.experimental.pallas import tpu as pltpu

_NE = 8
_R = 64
_SCALING = 32.0 / 64.0
_BMG = 512
_BM = 2048
_BN = 256
_NEG = -1e30


def _gate_body(x_ref, wg_ref, aall_ref, x16_ref, hs_ref, *, ne, r):
    xb = x_ref[...].astype(jnp.bfloat16)
    x16_ref[...] = xb
    ner = ne * r
    bm = xb.shape[0]
    logits = jax.lax.dot_general(
        xb, wg_ref[...], (((1,), (1,)), ((), ())),
        preferred_element_type=jnp.float32)
    idx = jax.lax.broadcasted_iota(jnp.int32, (bm, ne), 1)
    m1 = jnp.max(logits, axis=1, keepdims=True)
    am1 = jnp.min(jnp.where(logits == m1, idx, ne), axis=1, keepdims=True)
    oh1 = idx == am1
    neg = jnp.where(oh1, _NEG, logits)
    m2 = jnp.max(neg, axis=1, keepdims=True)
    am2 = jnp.min(jnp.where(neg == m2, idx, ne), axis=1, keepdims=True)
    oh2 = idx == am2
    p1 = 1.0 / (1.0 + jnp.exp(m2 - m1))
    p2 = 1.0 - p1
    w = jnp.where(oh1, p1, 0.0) + jnp.where(oh2, p2, 0.0)
    col_e = jax.lax.broadcasted_iota(jnp.int32, (ne, ner), 1) // r
    row_e = jax.lax.broadcasted_iota(jnp.int32, (ne, ner), 0)
    expand = (col_e == row_e).astype(jnp.float32)
    wexp = jnp.dot(w * _SCALING, expand,
                   preferred_element_type=jnp.float32)
    h = jax.lax.dot_general(
        xb, aall_ref[...], (((1,), (1,)), ((), ())),
        preferred_element_type=jnp.float32)
    hs_ref[...] = (h * wexp).astype(jnp.bfloat16)


def _mm_body(x16_ref, wb_ref, hs_ref, bcat_ref, out_ref):
    acc = jax.lax.dot_general(
        x16_ref[...], wb_ref[...].astype(jnp.bfloat16),
        (((1,), (1,)), ((), ())),
        preferred_element_type=jnp.float32)
    acc += jnp.dot(hs_ref[...], bcat_ref[...],
                   preferred_element_type=jnp.float32)
    out_ref[...] = acc


@functools.partial(jax.jit, static_argnames=("bmg", "bm", "bn", "interpret"))
def _mixlora(x, wb, wg, aall, bcat, bmg=_BMG, bm=_BM, bn=_BN,
             interpret=False):
    ntok, d = x.shape
    ne = wg.shape[0]
    ner = aall.shape[0]
    r = ner // ne
    x16, hs = pl.pallas_call(
        functools.partial(_gate_body, ne=ne, r=r),
        grid=(ntok // bmg,),
        in_specs=[
            pl.BlockSpec((bmg, d), lambda m: (m, 0)),
            pl.BlockSpec((ne, d), lambda m: (0, 0)),
            pl.BlockSpec((ner, d), lambda m: (0, 0)),
        ],
        out_specs=[
            pl.BlockSpec((bmg, d), lambda m: (m, 0)),
            pl.BlockSpec((bmg, ner), lambda m: (m, 0)),
        ],
        out_shape=[
            jax.ShapeDtypeStruct((ntok, d), jnp.bfloat16),
            jax.ShapeDtypeStruct((ntok, ner), jnp.bfloat16),
        ],
        compiler_params=pltpu.CompilerParams(
            dimension_semantics=("arbitrary",)),
        interpret=interpret,
    )(x, wg, aall)
    return pl.pallas_call(
        _mm_body,
        grid=(ntok // bm, d // bn),
        in_specs=[
            pl.BlockSpec((bm, d), lambda m, n: (m, 0)),
            pl.BlockSpec((bn, d), lambda m, n: (n, 0)),
            pl.BlockSpec((bm, ner), lambda m, n: (m, 0),
                         pipeline_mode=pl.Buffered(buffer_count=1)),
            pl.BlockSpec((ner, bn), lambda m, n: (0, n)),
        ],
        out_specs=pl.BlockSpec((bm, bn), lambda m, n: (m, n)),
        out_shape=jax.ShapeDtypeStruct((ntok, d), jnp.float32),
        compiler_params=pltpu.CompilerParams(
            dimension_semantics=("arbitrary", "arbitrary")),
        interpret=interpret,
    )(x16, wb, hs, bcat)


def kernel(x, W_base, W_gate, A, B):
    ne, r, d = A.shape
    wb = W_base
    wg = W_gate.astype(jnp.bfloat16)
    aall = A.reshape(ne * r, d).astype(jnp.bfloat16)
    bcat = B.transpose(0, 2, 1).reshape(ne * r, d).astype(jnp.bfloat16)
    return _mixlora(x, wb, wg, aall, bcat)
